# Initial kernel scaffold; baseline (speedup 1.0000x reference)
import functools

import jax
import jax.numpy as jnp
from jax import lax
from jax.experimental import pallas as pl
from jax.experimental.pallas import tpu as pltpu
from jax.experimental.pallas import tpu_sc as plsc

NC = 2
NS = 16
NW = NC * NS
CHUNK = 128

_mesh = plsc.VectorSubcoreMesh(core_axis_name="c", subcore_axis_name="s")


def _round_up(a, m):
    return ((a + m - 1) // m) * m


def _make_count(npad, epad):
    per_w = epad // NW
    n_chunks = per_w // CHUNK
    stripe = npad // NS

    @functools.partial(
        pl.kernel,
        out_type=jax.ShapeDtypeStruct((NC, npad, 16), jnp.float32),
        mesh=_mesh,
        scratch_types=[
            pltpu.VMEM((CHUNK,), jnp.int32),
            pltpu.VMEM((CHUNK, 16), jnp.float32),
            pltpu.VMEM_SHARED((npad, 16), jnp.float32),
        ],
    )
    def count(row_hbm, out_hbm, ridx_v, ones_v, acc):
        cid = lax.axis_index("c")
        sid = lax.axis_index("s")
        wid = sid * NC + cid

        @pl.loop(0, CHUNK)
        def _(i):
            ones_v[pl.ds(i, 1), :] = jnp.zeros((1, 16), jnp.float32)

        @pl.loop(0, stripe // CHUNK)
        def _(t):
            pltpu.sync_copy(ones_v, acc.at[pl.ds(sid * stripe + t * CHUNK, CHUNK)])

        @pl.loop(0, CHUNK)
        def _(i):
            ones_v[pl.ds(i, 1), :] = jnp.full((1, 16), 0.0625, jnp.float32)

        plsc.subcore_barrier()

        base = wid * per_w

        @pl.loop(0, n_chunks)
        def _(c):
            pltpu.sync_copy(row_hbm.at[pl.ds(base + c * CHUNK, CHUNK)], ridx_v)
            pltpu.sync_copy(ones_v, acc.at[ridx_v], add=True)

        plsc.subcore_barrier()

        @pl.loop(0, stripe // CHUNK)
        def _(t):
            r0 = sid * stripe + t * CHUNK
            pltpu.sync_copy(acc.at[pl.ds(r0, CHUNK)], out_hbm.at[cid, pl.ds(r0, CHUNK)])

    return count


def _make_scatter(npad, epad, d):
    per_w = epad // NW
    n_chunks = per_w // CHUNK
    stripe = npad // NS

    @functools.partial(
        pl.kernel,
        out_type=jax.ShapeDtypeStruct((NC, npad, d), jnp.float32),
        mesh=_mesh,
        scratch_types=[
            pltpu.VMEM((CHUNK,), jnp.int32),
            pltpu.VMEM((CHUNK,), jnp.int32),
            pltpu.VMEM((CHUNK, d), jnp.float32),
            pltpu.VMEM_SHARED((npad, d), jnp.float32),
        ],
    )
    def scatter(h_hbm, row_hbm, col_hbm, out_hbm, ridx_v, cidx_v, rows_v, acc):
        cid = lax.axis_index("c")
        sid = lax.axis_index("s")
        wid = sid * NC + cid

        @pl.loop(0, CHUNK)
        def _(i):
            @pl.loop(0, d // 16)
            def _(j):
                rows_v[pl.ds(i, 1), pl.ds(j * 16, 16)] = jnp.zeros((1, 16), jnp.float32)

        @pl.loop(0, stripe // CHUNK)
        def _(t):
            pltpu.sync_copy(rows_v, acc.at[pl.ds(sid * stripe + t * CHUNK, CHUNK)])

        plsc.subcore_barrier()

        base = wid * per_w

        @pl.loop(0, n_chunks)
        def _(c):
            off = base + c * CHUNK
            pltpu.sync_copy(row_hbm.at[pl.ds(off, CHUNK)], ridx_v)
            pltpu.sync_copy(col_hbm.at[pl.ds(off, CHUNK)], cidx_v)
            pltpu.sync_copy(h_hbm.at[ridx_v], rows_v)
            pltpu.sync_copy(rows_v, acc.at[cidx_v], add=True)

        plsc.subcore_barrier()

        @pl.loop(0, stripe // CHUNK)
        def _(t):
            r0 = sid * stripe + t * CHUNK
            pltpu.sync_copy(acc.at[pl.ds(r0, CHUNK)], out_hbm.at[cid, pl.ds(r0, CHUNK)])

    return scatter


def _tc_matmul(x, w):
    def body(x_ref, w_ref, o_ref):
        o_ref[...] = jnp.dot(x_ref[...], w_ref[...],
                             preferred_element_type=jnp.float32)

    return pl.pallas_call(
        body,
        out_shape=jax.ShapeDtypeStruct((x.shape[0], w.shape[1]), jnp.float32),
    )(x, w)


def _dis(degp_ref):
    deg = jnp.sum(degp_ref[0] + degp_ref[1], axis=1, keepdims=True) + 1.0
    return lax.rsqrt(deg)


def _tc_scale(xw, degp):
    def body(xw_ref, degp_ref, o_ref):
        o_ref[...] = _dis(degp_ref) * xw_ref[...]

    return pl.pallas_call(
        body,
        out_shape=jax.ShapeDtypeStruct(xw.shape, jnp.float32),
    )(xw, degp)


def _tc_layer(sp, hp, degp, b, w):
    def body(sp_ref, hp_ref, degp_ref, b_ref, w_ref, o_ref):
        dis = _dis(degp_ref)
        aggr = dis * (sp_ref[0] + sp_ref[1] + hp_ref[...]) + b_ref[...]
        h = jnp.maximum(aggr, 0.0)
        o_ref[...] = dis * jnp.dot(h, w_ref[...],
                                   preferred_element_type=jnp.float32)

    return pl.pallas_call(
        body,
        out_shape=jax.ShapeDtypeStruct((hp.shape[0], w.shape[1]), jnp.float32),
    )(sp, hp, degp, b, w)


def _tc_finish(sp, hp, degp, b):
    def body(sp_ref, hp_ref, degp_ref, b_ref, o_ref):
        dis = _dis(degp_ref)
        o_ref[...] = dis * (sp_ref[0] + sp_ref[1] + hp_ref[...]) + b_ref[...]

    return pl.pallas_call(
        body,
        out_shape=jax.ShapeDtypeStruct(hp.shape, jnp.float32),
    )(sp, hp, degp, b)


def kernel(x, edge_index, W1, b1, W2, b2):
    n, in_ch = x.shape
    e = edge_index.shape[1]
    npad = _round_up(n + 1, 16 * CHUNK)
    epad = _round_up(e, NW * CHUNK)

    xpad = jnp.zeros((npad, in_ch), x.dtype).at[:n].set(x)
    pad_idx = jnp.full((epad - e,), n, jnp.int32)
    rowp = jnp.concatenate([edge_index[0], pad_idx])
    colp = jnp.concatenate([edge_index[1], pad_idx])
    b1r = b1.reshape(1, -1)
    b2r = b2.reshape(1, -1)

    count = _make_count(npad, epad)
    scat1 = _make_scatter(npad, epad, W1.shape[1])
    scat2 = _make_scatter(npad, epad, W2.shape[1])

    degp = count(rowp)
    xw = _tc_matmul(xpad, W1)
    hp1 = _tc_scale(xw, degp)
    s1 = scat1(hp1, rowp, colp)
    hp2 = _tc_layer(s1, hp1, degp, b1r, W2)
    s2 = scat2(hp2, rowp, colp)
    outp = _tc_finish(s2, hp2, degp, b2r)
    return outp[:n]


# speedup vs baseline: 10.3308x; 10.3308x over previous
import functools

import jax
import jax.numpy as jnp
from jax import lax
from jax.experimental import pallas as pl
from jax.experimental.pallas import tpu as pltpu
from jax.experimental.pallas import tpu_sc as plsc

NC = 2
NS = 16
NW = NC * NS
CHUNK = 128

_mesh = plsc.VectorSubcoreMesh(core_axis_name="c", subcore_axis_name="s")


def _round_up(a, m):
    return ((a + m - 1) // m) * m


def _make_count(npad, epad):
    per_w = epad // NW
    n_chunks = per_w // CHUNK
    stripe = npad // NS
    d = 128

    @functools.partial(
        pl.kernel,
        out_type=jax.ShapeDtypeStruct((NC, npad, d), jnp.float32),
        mesh=_mesh,
        scratch_types=[
            pltpu.VMEM((CHUNK,), jnp.int32),
            pltpu.VMEM((CHUNK, d), jnp.float32),
            pltpu.VMEM_SHARED((npad, d), jnp.float32),
        ],
    )
    def count(row_hbm, out_hbm, ridx_v, ones_v, acc):
        cid = lax.axis_index("c")
        sid = lax.axis_index("s")
        wid = sid * NC + cid

        @pl.loop(0, CHUNK)
        def _(i):
            @pl.loop(0, d // 16)
            def _(j):
                ones_v[pl.ds(i, 1), pl.ds(j * 16, 16)] = jnp.zeros((1, 16), jnp.float32)

        @pl.loop(0, stripe // CHUNK)
        def _(t):
            pltpu.sync_copy(ones_v, acc.at[pl.ds(sid * stripe + t * CHUNK, CHUNK)])

        @pl.loop(0, CHUNK)
        def _(i):
            @pl.loop(0, d // 16)
            def _(j):
                ones_v[pl.ds(i, 1), pl.ds(j * 16, 16)] = jnp.full(
                    (1, 16), 1.0 / d, jnp.float32)

        plsc.subcore_barrier()

        base = wid * per_w

        @pl.loop(0, n_chunks)
        def _(c):
            pltpu.sync_copy(row_hbm.at[pl.ds(base + c * CHUNK, CHUNK)], ridx_v)
            pltpu.sync_copy(ones_v, acc.at[ridx_v], add=True)

        plsc.subcore_barrier()

        @pl.loop(0, stripe // CHUNK)
        def _(t):
            r0 = sid * stripe + t * CHUNK
            pltpu.sync_copy(acc.at[pl.ds(r0, CHUNK)], out_hbm.at[cid, pl.ds(r0, CHUNK)])

    return count


def _make_scatter(npad, epad, d):
    per_w = epad // NW
    n_chunks = per_w // CHUNK
    stripe = npad // NS

    @functools.partial(
        pl.kernel,
        out_type=jax.ShapeDtypeStruct((NC, npad, d), jnp.float32),
        mesh=_mesh,
        scratch_types=[
            pltpu.VMEM((CHUNK,), jnp.int32),
            pltpu.VMEM((CHUNK,), jnp.int32),
            pltpu.VMEM((CHUNK, d), jnp.float32),
            pltpu.VMEM_SHARED((npad, d), jnp.float32),
        ],
    )
    def scatter(h_hbm, row_hbm, col_hbm, out_hbm, ridx_v, cidx_v, rows_v, acc):
        cid = lax.axis_index("c")
        sid = lax.axis_index("s")
        wid = sid * NC + cid

        @pl.loop(0, CHUNK)
        def _(i):
            @pl.loop(0, d // 16)
            def _(j):
                rows_v[pl.ds(i, 1), pl.ds(j * 16, 16)] = jnp.zeros((1, 16), jnp.float32)

        @pl.loop(0, stripe // CHUNK)
        def _(t):
            pltpu.sync_copy(rows_v, acc.at[pl.ds(sid * stripe + t * CHUNK, CHUNK)])

        plsc.subcore_barrier()

        base = wid * per_w

        @pl.loop(0, n_chunks)
        def _(c):
            off = base + c * CHUNK
            pltpu.sync_copy(row_hbm.at[pl.ds(off, CHUNK)], ridx_v)
            pltpu.sync_copy(col_hbm.at[pl.ds(off, CHUNK)], cidx_v)
            pltpu.sync_copy(h_hbm.at[ridx_v], rows_v)
            pltpu.sync_copy(rows_v, acc.at[cidx_v], add=True)

        plsc.subcore_barrier()

        @pl.loop(0, stripe // CHUNK)
        def _(t):
            r0 = sid * stripe + t * CHUNK
            pltpu.sync_copy(acc.at[pl.ds(r0, CHUNK)], out_hbm.at[cid, pl.ds(r0, CHUNK)])

    return scatter


def _tc_matmul(x, w):
    def body(x_ref, w_ref, o_ref):
        o_ref[...] = jnp.dot(x_ref[...], w_ref[...],
                             preferred_element_type=jnp.float32)

    return pl.pallas_call(
        body,
        out_shape=jax.ShapeDtypeStruct((x.shape[0], w.shape[1]), jnp.float32),
    )(x, w)


def _dis(degp_ref):
    deg = jnp.sum(degp_ref[0] + degp_ref[1], axis=1, keepdims=True) + 1.0
    return lax.rsqrt(deg)


def _tc_scale(xw, degp):
    def body(xw_ref, degp_ref, o_ref):
        o_ref[...] = _dis(degp_ref) * xw_ref[...]

    return pl.pallas_call(
        body,
        out_shape=jax.ShapeDtypeStruct(xw.shape, jnp.float32),
    )(xw, degp)


def _tc_layer(sp, hp, degp, b, w):
    def body(sp_ref, hp_ref, degp_ref, b_ref, w_ref, o_ref):
        dis = _dis(degp_ref)
        aggr = dis * (sp_ref[0] + sp_ref[1] + hp_ref[...]) + b_ref[...]
        h = jnp.maximum(aggr, 0.0)
        o_ref[...] = dis * jnp.dot(h, w_ref[...],
                                   preferred_element_type=jnp.float32)

    return pl.pallas_call(
        body,
        out_shape=jax.ShapeDtypeStruct((hp.shape[0], w.shape[1]), jnp.float32),
    )(sp, hp, degp, b, w)


def _tc_finish(sp, hp, degp, b):
    def body(sp_ref, hp_ref, degp_ref, b_ref, o_ref):
        dis = _dis(degp_ref)
        o_ref[...] = dis * (sp_ref[0] + sp_ref[1] + hp_ref[...]) + b_ref[...]

    return pl.pallas_call(
        body,
        out_shape=jax.ShapeDtypeStruct(hp.shape, jnp.float32),
    )(sp, hp, degp, b)


def kernel(x, edge_index, W1, b1, W2, b2):
    n, in_ch = x.shape
    e = edge_index.shape[1]
    npad = _round_up(n + 1, 16 * CHUNK)
    epad = _round_up(e, NW * CHUNK)

    out_ch = W2.shape[1]
    oc_pad = _round_up(out_ch, 128)

    xpad = jnp.zeros((npad, in_ch), x.dtype).at[:n].set(x)
    pad_idx = jnp.full((epad - e,), n, jnp.int32)
    rowp = jnp.concatenate([edge_index[0], pad_idx])
    colp = jnp.concatenate([edge_index[1], pad_idx])
    b1r = b1.reshape(1, -1)
    b2r = jnp.zeros((1, oc_pad), b2.dtype).at[0, :out_ch].set(b2)
    W2p = jnp.zeros((W2.shape[0], oc_pad), W2.dtype).at[:, :out_ch].set(W2)

    count = _make_count(npad, epad)
    scat1 = _make_scatter(npad, epad, W1.shape[1])
    scat2 = _make_scatter(npad, epad, oc_pad)

    degp = count(rowp)
    xw = _tc_matmul(xpad, W1)
    hp1 = _tc_scale(xw, degp)
    s1 = scat1(hp1, rowp, colp)
    hp2 = _tc_layer(s1, hp1, degp, b1r, W2p)
    s2 = scat2(hp2, rowp, colp)
    outp = _tc_finish(s2, hp2, degp, b2r)
    return outp[:n, :out_ch]
